# baseline (speedup 1.0000x reference)
import functools

import jax
import jax.numpy as jnp
from jax import lax
from jax.experimental import pallas as pl
from jax.experimental.pallas import tpu as pltpu
from jax.experimental.pallas import tpu_sc as plsc

BN_EPS = 1e-5


def _dense_tail(hnew, w_ref, wl_ref, wr_ref, zelr_ref, era_ref, cm_ref):
    z = jnp.dot(hnew, w_ref[...], preferred_element_type=jnp.float32)
    el = jnp.dot(z, wl_ref[...], preferred_element_type=jnp.float32)
    er = jnp.dot(z, wr_ref[...], preferred_element_type=jnp.float32)
    zelr_ref[...] = jnp.concatenate([z, el, er], axis=1)
    era_ref[...] = jnp.concatenate([er, jnp.zeros_like(er)], axis=1)
    bm = jnp.stack([jnp.max(el, axis=0), jnp.max(er, axis=0)])

    @pl.when(pl.program_id(0) == 0)
    def _():
        cm_ref[...] = bm

    @pl.when(pl.program_id(0) != 0)
    def _():
        cm_ref[...] = jnp.maximum(cm_ref[...], bm)


def _first_body(x_ref, wemb_ref, bemb_ref, w_ref, wl_ref, wr_ref,
                zelr_ref, era_ref, cm_ref, h_ref):
    h = jnp.dot(x_ref[...], wemb_ref[...],
                preferred_element_type=jnp.float32) + bemb_ref[...]
    h_ref[...] = h
    _dense_tail(h, w_ref, wl_ref, wr_ref, zelr_ref, era_ref, cm_ref)


def _merge_head(u0_ref, u1_ref, p_ref, gs_ref, bt_ref, hp_ref):
    ud = u0_ref[...] + u1_ref[...]
    u = ud[:, :128]
    den = ud[:, 128:] + 1e-16
    dexp = jnp.dot(den, p_ref[...], preferred_element_type=jnp.float32)
    agg = u / dexp
    neg = jnp.exp(jnp.minimum(agg, 0.0)) - 1.0
    out = jnp.where(agg > 0.0, agg, neg)
    return out * gs_ref[...] + bt_ref[...] + hp_ref[...]


def _mid_body(u0_ref, u1_ref, p_ref, gs_ref, bt_ref, hp_ref,
              w_ref, wl_ref, wr_ref, zelr_ref, era_ref, cm_ref, h_ref):
    h = _merge_head(u0_ref, u1_ref, p_ref, gs_ref, bt_ref, hp_ref)
    h_ref[...] = h
    _dense_tail(h, w_ref, wl_ref, wr_ref, zelr_ref, era_ref, cm_ref)


def _last_body(u0_ref, u1_ref, p_ref, gs_ref, bt_ref, hp_ref, o_ref):
    o_ref[...] = _merge_head(u0_ref, u1_ref, p_ref, gs_ref, bt_ref, hp_ref)


_RB = 2000


def _row_spec(cols):
    return pl.BlockSpec((_RB, cols), lambda i: (i, 0))


def _full_spec(r, cols):
    return pl.BlockSpec((r, cols), lambda i: (0, 0))


def _tail_out(n):
    return (
        [_row_spec(144), _row_spec(16), _full_spec(2, 8), _row_spec(128)],
        [jax.ShapeDtypeStruct((n, 144), jnp.float32),
         jax.ShapeDtypeStruct((n, 16), jnp.float32),
         jax.ShapeDtypeStruct((2, 8), jnp.float32),
         jax.ShapeDtypeStruct((n, 128), jnp.float32)],
    )


def _tc_first(x, wemb, bemb, w, wl, wr):
    n = x.shape[0]
    out_specs, out_shape = _tail_out(n)
    return pl.pallas_call(
        _first_body,
        grid=(n // _RB,),
        in_specs=[
            _row_spec(128), _full_spec(128, 128), _full_spec(1, 128),
            _full_spec(128, 128), _full_spec(128, 8), _full_spec(128, 8),
        ],
        out_specs=out_specs,
        out_shape=out_shape,
    )(x, wemb, bemb.reshape(1, 128), w, wl, wr)


def _tc_mid(u2, p, gs, bt, hp, w, wl, wr):
    n = hp.shape[0]
    out_specs, out_shape = _tail_out(n)
    return pl.pallas_call(
        _mid_body,
        grid=(n // _RB,),
        in_specs=[
            _row_spec(136), _row_spec(136), _full_spec(8, 128),
            _full_spec(1, 128), _full_spec(1, 128), _row_spec(128),
            _full_spec(128, 128), _full_spec(128, 8), _full_spec(128, 8),
        ],
        out_specs=out_specs,
        out_shape=out_shape,
    )(u2[0], u2[1], p, gs, bt, hp, w, wl, wr)


def _tc_last(u2, p, gs, bt, hp):
    n = hp.shape[0]
    return pl.pallas_call(
        _last_body,
        grid=(n // _RB,),
        in_specs=[
            _row_spec(136), _row_spec(136), _full_spec(8, 128),
            _full_spec(1, 128), _full_spec(1, 128), _row_spec(128),
        ],
        out_specs=_row_spec(128),
        out_shape=jax.ShapeDtypeStruct((n, 128), jnp.float32),
    )(u2[0], u2[1], p, gs, bt, hp)


_B = 80
_GDN = lax.GatherDimensionNumbers(offset_dims=(), collapsed_slice_dims=(0,),
                                  start_index_map=(0,))


def _sc_edge_body(heads, n, e, zelr, era, c2, src, dst, z136,
                  u2,
                  u_acc, srcv0, srcv1, dstv0, dstv1,
                  zelg0, zelg1, erg0, erg1, wz, cexp,
                  sg0, sg1, si0, si1):
    nsub = 16
    rows_per = 1000
    edges_per = e // (2 * nsub)
    nblk = edges_per // _B
    c = lax.axis_index("c")
    s = lax.axis_index("s")
    r0 = s * rows_per
    srcv = (srcv0, srcv1)
    dstv = (dstv0, dstv1)
    zelg = (zelg0, zelg1)
    erg = (erg0, erg1)
    sg = (sg0, sg1)
    si = (si0, si1)
    ebase = (c * nsub + s) * edges_per

    @pl.when(s < n // rows_per)
    def _():
        pltpu.sync_copy(z136, u_acc.at[pl.ds(r0, rows_per)])

    pltpu.sync_copy(c2, cexp)

    plsc.subcore_barrier()

    iota = lax.iota(jnp.int32, 16)
    hmap = [hh if heads > 1 else 0 for hh in range(8)]

    def issue_idx(bi, p):
        base = ebase + bi * _B
        pltpu.async_copy(src.at[pl.ds(base, _B)], srcv[p], si[p])
        pltpu.async_copy(dst.at[pl.ds(base, _B)], dstv[p], si[p])

    def drain_idx(p):
        pltpu.make_async_copy(src.at[pl.ds(0, _B)], srcv[p], si[p]).wait()
        pltpu.make_async_copy(dst.at[pl.ds(0, _B)], dstv[p], si[p]).wait()

    def issue_gathers(p):
        pltpu.async_copy(zelr.at[srcv[p]], zelg[p], sg[p])
        pltpu.async_copy(era.at[dstv[p]], erg[p], sg[p])

    def drain_gathers(p):
        pltpu.make_async_copy(zelr.at[pl.ds(0, _B)], zelg[p], sg[p]).wait()
        pltpu.make_async_copy(era.at[pl.ds(0, _B)], erg[p], sg[p]).wait()

    cvv = cexp[...]
    erow = iota // 8
    hcol = iota % 8

    def compute_block(p):
        zg = zelg[p]
        eg = erg[p]
        dv = dstv[p]

        def fuse_blk(k, carry2):
            rows = erow + 2 * k
            x = (plsc.load_gather(zg, [rows, 128 + hcol])
                 + plsc.load_gather(eg, [rows, hcol]))
            x = jnp.maximum(x, 0.2 * x)
            v = jnp.exp(x - cvv)
            plsc.store_scatter(wz, [rows, 128 + hcol], v)
            for e01 in range(2):
                ei = 2 * k + e01
                for h in range(8):
                    lane = jnp.full((16, 1), 8 * e01 + hmap[h], jnp.int32)
                    w = lax.gather(
                        v, lane, _GDN, slice_sizes=(1,),
                        mode=lax.GatherScatterMode.PROMISE_IN_BOUNDS)
                    wz[ei, pl.ds(16 * h, 16)] = w * zg[ei, pl.ds(16 * h, 16)]
            return carry2

        lax.fori_loop(0, _B // 2, fuse_blk, 0, unroll=2)
        pltpu.sync_copy(wz, u_acc.at[dv], add=True)

    issue_idx(0, 0)
    issue_idx(1, 1)
    drain_idx(0)
    issue_gathers(0)

    def grp(g, carry):
        for p in (0, 1):
            b = 2 * g + p
            drain_idx(1 - p)
            issue_gathers(1 - p)
            drain_gathers(p)
            compute_block(p)
            issue_idx(b + 2, p)
        return carry

    lax.fori_loop(0, (nblk - 3) // 2, grp, 0, unroll=False)
    drain_idx(1)
    issue_gathers(1)
    drain_gathers(0)
    compute_block(0)
    issue_idx(nblk - 1, 0)
    drain_idx(0)
    issue_gathers(0)
    drain_gathers(1)
    compute_block(1)
    drain_gathers(0)
    compute_block(0)

    plsc.subcore_barrier()

    @pl.when(s < n // rows_per)
    def _():
        pltpu.sync_copy(u_acc.at[pl.ds(r0, rows_per)],
                        u2.at[c, pl.ds(r0, rows_per)])


@functools.partial(jax.jit, static_argnums=(0,))
def _sc_edge(heads, zelr, era, c2, src, dst, z136):
    n = zelr.shape[0]
    e = src.shape[0]
    mesh = plsc.VectorSubcoreMesh(core_axis_name="c", subcore_axis_name="s",
                                  num_cores=2, num_subcores=16)
    body = functools.partial(_sc_edge_body, heads, n, e)
    f = pl.kernel(
        body,
        out_type=[
            jax.ShapeDtypeStruct((2, n, 136), jnp.float32),
        ],
        mesh=mesh,
        compiler_params=pltpu.CompilerParams(use_tc_tiling_on_sc=False, needs_layout_passes=False),
        scratch_types=[
            pltpu.VMEM_SHARED((n, 136), jnp.float32),
            pltpu.VMEM((_B,), jnp.int32),
            pltpu.VMEM((_B,), jnp.int32),
            pltpu.VMEM((_B,), jnp.int32),
            pltpu.VMEM((_B,), jnp.int32),
            pltpu.VMEM((_B, 144), jnp.float32),
            pltpu.VMEM((_B, 144), jnp.float32),
            pltpu.VMEM((_B, 16), jnp.float32),
            pltpu.VMEM((_B, 16), jnp.float32),
            pltpu.VMEM((_B, 136), jnp.float32),
            pltpu.VMEM((16,), jnp.float32),
            pltpu.SemaphoreType.DMA,
            pltpu.SemaphoreType.DMA,
            pltpu.SemaphoreType.DMA,
            pltpu.SemaphoreType.DMA,
        ],
    )
    return f(zelr, era, c2, src, dst, z136)


def _expand_att(a):
    heads, outd = a.shape
    k = jnp.arange(128)
    m = (k[:, None] // outd == jnp.arange(8)[None, :]).astype(jnp.float32)
    return m * a.reshape(-1)[:, None]


def _expand_p(outd):
    return (jnp.arange(8)[:, None] == (jnp.arange(128)[None, :] // outd)
            ).astype(jnp.float32)


def kernel(feature, edge_index, W_emb, b_emb, W1, al1, ar1, g1, bt1,
           W2, al2, ar2, g2, bt2, W3, al3, ar3, g3, bt3,
           W4, al4, ar4, g4, bt4):
    n = feature.shape[0]
    src = edge_index[0]
    dst = edge_index[1]
    z136 = jnp.zeros((1000, 136), jnp.float32)
    bn_scale = 1.0 / jnp.sqrt(1.0 + BN_EPS)

    layers = [(W1, al1, ar1, g1, bt1, 8), (W2, al2, ar2, g2, bt2, 8),
              (W3, al3, ar3, g3, bt3, 8), (W4, al4, ar4, g4, bt4, 1)]
    zelr, era, cm, h = _tc_first(feature, W_emb, b_emb.reshape(1, 128),
                                 W1, _expand_att(al1), _expand_att(ar1))
    for i, (w, al, ar, g, bt, heads) in enumerate(layers):
        outd = 128 // heads
        cc = cm[0] + cm[1]
        shift = jnp.maximum(cc, 0.2 * cc)
        c2 = jnp.tile(shift, 2)
        (u2,) = _sc_edge(heads, zelr, era, c2, src, dst, z136)
        gs = (g * bn_scale).reshape(1, 128)
        btr = bt.reshape(1, 128)
        if i < 3:
            nw, nal, nar, _, _, _ = layers[i + 1]
            zelr, era, cm, h = _tc_mid(u2, _expand_p(outd), gs, btr, h,
                                       nw, _expand_att(nal), _expand_att(nar))
        else:
            h = _tc_last(u2, _expand_p(outd), gs, btr, h)
    return h


# speedup vs baseline: 1.0867x; 1.0867x over previous
import functools

import jax
import jax.numpy as jnp
from jax import lax
from jax.experimental import pallas as pl
from jax.experimental.pallas import tpu as pltpu
from jax.experimental.pallas import tpu_sc as plsc

BN_EPS = 1e-5


def _dense_tail(hnew, w_ref, wl_ref, wr_ref, zelr_ref, era_ref, cm_ref):
    z = jnp.dot(hnew, w_ref[...], preferred_element_type=jnp.float32)
    el = jnp.dot(z, wl_ref[...], preferred_element_type=jnp.float32)
    er = jnp.dot(z, wr_ref[...], preferred_element_type=jnp.float32)
    zelr_ref[...] = jnp.concatenate([z, el], axis=1)
    era_ref[...] = er
    bm = jnp.stack([jnp.max(el, axis=0), jnp.max(er, axis=0)])

    @pl.when(pl.program_id(0) == 0)
    def _():
        cm_ref[...] = bm

    @pl.when(pl.program_id(0) != 0)
    def _():
        cm_ref[...] = jnp.maximum(cm_ref[...], bm)


def _first_body(x_ref, wemb_ref, bemb_ref, w_ref, wl_ref, wr_ref,
                zelr_ref, era_ref, cm_ref, h_ref):
    h = jnp.dot(x_ref[...], wemb_ref[...],
                preferred_element_type=jnp.float32) + bemb_ref[...]
    h_ref[...] = h
    _dense_tail(h, w_ref, wl_ref, wr_ref, zelr_ref, era_ref, cm_ref)


def _merge_head(u0_ref, u1_ref, p_ref, gs_ref, bt_ref, hp_ref):
    ud = u0_ref[...] + u1_ref[...]
    u = ud[:, :128]
    den = ud[:, 128:] + 1e-16
    dexp = jnp.dot(den, p_ref[...], preferred_element_type=jnp.float32)
    agg = u / dexp
    neg = jnp.exp(jnp.minimum(agg, 0.0)) - 1.0
    out = jnp.where(agg > 0.0, agg, neg)
    return out * gs_ref[...] + bt_ref[...] + hp_ref[...]


def _mid_body(u0_ref, u1_ref, p_ref, gs_ref, bt_ref, hp_ref,
              w_ref, wl_ref, wr_ref, zelr_ref, era_ref, cm_ref, h_ref):
    h = _merge_head(u0_ref, u1_ref, p_ref, gs_ref, bt_ref, hp_ref)
    h_ref[...] = h
    _dense_tail(h, w_ref, wl_ref, wr_ref, zelr_ref, era_ref, cm_ref)


def _last_body(u0_ref, u1_ref, p_ref, gs_ref, bt_ref, hp_ref, o_ref):
    o_ref[...] = _merge_head(u0_ref, u1_ref, p_ref, gs_ref, bt_ref, hp_ref)


_RB = 2000


def _row_spec(cols):
    return pl.BlockSpec((_RB, cols), lambda i: (i, 0))


def _full_spec(r, cols):
    return pl.BlockSpec((r, cols), lambda i: (0, 0))


def _tail_out(n):
    return (
        [_row_spec(136), _row_spec(8), _full_spec(2, 8), _row_spec(128)],
        [jax.ShapeDtypeStruct((n, 136), jnp.float32),
         jax.ShapeDtypeStruct((n, 8), jnp.float32),
         jax.ShapeDtypeStruct((2, 8), jnp.float32),
         jax.ShapeDtypeStruct((n, 128), jnp.float32)],
    )


def _tc_first(x, wemb, bemb, w, wl, wr):
    n = x.shape[0]
    out_specs, out_shape = _tail_out(n)
    return pl.pallas_call(
        _first_body,
        grid=(n // _RB,),
        in_specs=[
            _row_spec(128), _full_spec(128, 128), _full_spec(1, 128),
            _full_spec(128, 128), _full_spec(128, 8), _full_spec(128, 8),
        ],
        out_specs=out_specs,
        out_shape=out_shape,
    )(x, wemb, bemb.reshape(1, 128), w, wl, wr)


def _tc_mid(u2, p, gs, bt, hp, w, wl, wr):
    n = hp.shape[0]
    out_specs, out_shape = _tail_out(n)
    return pl.pallas_call(
        _mid_body,
        grid=(n // _RB,),
        in_specs=[
            _row_spec(136), _row_spec(136), _full_spec(8, 128),
            _full_spec(1, 128), _full_spec(1, 128), _row_spec(128),
            _full_spec(128, 128), _full_spec(128, 8), _full_spec(128, 8),
        ],
        out_specs=out_specs,
        out_shape=out_shape,
    )(u2[0], u2[1], p, gs, bt, hp, w, wl, wr)


def _tc_last(u2, p, gs, bt, hp):
    n = hp.shape[0]
    return pl.pallas_call(
        _last_body,
        grid=(n // _RB,),
        in_specs=[
            _row_spec(136), _row_spec(136), _full_spec(8, 128),
            _full_spec(1, 128), _full_spec(1, 128), _row_spec(128),
        ],
        out_specs=_row_spec(128),
        out_shape=jax.ShapeDtypeStruct((n, 128), jnp.float32),
    )(u2[0], u2[1], p, gs, bt, hp)


_B = 80
_GDN = lax.GatherDimensionNumbers(offset_dims=(), collapsed_slice_dims=(0,),
                                  start_index_map=(0,))


def _sc_edge_body(heads, n, e, zelr, era, c2, src, dst, z136,
                  u2,
                  u_acc, srcv0, srcv1, dstv0, dstv1,
                  zelg0, zelg1, erg0, erg1, wz0, wz1, dsc0, dsc1, cexp,
                  sg0, sg1, si0, si1, ws0, ws1):
    nsub = 16
    rows_per = 1000
    edges_per = e // (2 * nsub)
    nblk = edges_per // _B
    c = lax.axis_index("c")
    s = lax.axis_index("s")
    r0 = s * rows_per
    srcv = (srcv0, srcv1)
    dstv = (dstv0, dstv1)
    zelg = (zelg0, zelg1)
    erg = (erg0, erg1)
    sg = (sg0, sg1)
    si = (si0, si1)
    wzb = (wz0, wz1)
    dsc = (dsc0, dsc1)
    ws = (ws0, ws1)
    ebase = (c * nsub + s) * edges_per

    @pl.when(s < n // rows_per)
    def _():
        pltpu.sync_copy(z136, u_acc.at[pl.ds(r0, rows_per)])

    pltpu.sync_copy(c2, cexp)

    plsc.subcore_barrier()

    for p in (0, 1):
        pltpu.sync_copy(z136.at[pl.ds(0, _B)], wzb[p])
        pltpu.sync_copy(src.at[pl.ds(0, _B)], dsc[p])
        pltpu.async_copy(wzb[p], u_acc.at[dsc[p]], ws[p], add=True)

    iota = lax.iota(jnp.int32, 16)
    hmap = [hh if heads > 1 else 0 for hh in range(8)]

    def issue_idx(bi, p):
        base = ebase + bi * _B
        pltpu.async_copy(src.at[pl.ds(base, _B)], srcv[p], si[p])
        pltpu.async_copy(dst.at[pl.ds(base, _B)], dstv[p], si[p])

    def drain_idx(p):
        pltpu.make_async_copy(src.at[pl.ds(0, _B)], srcv[p], si[p]).wait()
        pltpu.make_async_copy(dst.at[pl.ds(0, _B)], dstv[p], si[p]).wait()

    def issue_gathers(p):
        pltpu.async_copy(zelr.at[srcv[p]], zelg[p], sg[p])
        pltpu.async_copy(era.at[dstv[p]], erg[p], sg[p])

    def drain_gathers(p):
        pltpu.make_async_copy(zelr.at[pl.ds(0, _B)], zelg[p], sg[p]).wait()
        pltpu.make_async_copy(era.at[pl.ds(0, _B)], erg[p], sg[p]).wait()

    cvv = cexp[...]
    erow = iota // 8
    hcol = iota % 8

    def compute_block(p):
        zg = zelg[p]
        eg = erg[p]
        dv = dstv[p]
        wz = wzb[p]

        pltpu.make_async_copy(wz, u_acc.at[pl.ds(0, _B)], ws[p]).wait()

        def fuse_blk(k, carry2):
            rows = erow + 2 * k
            x = (plsc.load_gather(zg, [rows, 128 + hcol])
                 + plsc.load_gather(eg, [rows, hcol]))
            x = jnp.maximum(x, 0.2 * x)
            v = jnp.exp(x - cvv)
            plsc.store_scatter(wz, [rows, 128 + hcol], v)
            for e01 in range(2):
                ei = 2 * k + e01
                for h in range(8):
                    lane = jnp.full((16, 1), 8 * e01 + hmap[h], jnp.int32)
                    w = lax.gather(
                        v, lane, _GDN, slice_sizes=(1,),
                        mode=lax.GatherScatterMode.PROMISE_IN_BOUNDS)
                    wz[ei, pl.ds(16 * h, 16)] = w * zg[ei, pl.ds(16 * h, 16)]
            return carry2

        lax.fori_loop(0, _B // 2, fuse_blk, 0, unroll=False)
        for j in range(_B // 16):
            dsc[p][pl.ds(16 * j, 16)] = dv[pl.ds(16 * j, 16)]
        pltpu.async_copy(wz, u_acc.at[dsc[p]], ws[p], add=True)

    issue_idx(0, 0)
    issue_idx(1, 1)
    drain_idx(0)
    issue_gathers(0)

    def grp(g, carry):
        for p in (0, 1):
            b = 2 * g + p
            drain_idx(1 - p)
            issue_gathers(1 - p)
            drain_gathers(p)
            compute_block(p)
            issue_idx(b + 2, p)
        return carry

    lax.fori_loop(0, (nblk - 3) // 2, grp, 0, unroll=False)
    drain_idx(1)
    issue_gathers(1)
    drain_gathers(0)
    compute_block(0)
    issue_idx(nblk - 1, 0)
    drain_idx(0)
    issue_gathers(0)
    drain_gathers(1)
    compute_block(1)
    drain_gathers(0)
    compute_block(0)

    for p in (0, 1):
        pltpu.make_async_copy(wzb[p], u_acc.at[pl.ds(0, _B)], ws[p]).wait()

    plsc.subcore_barrier()

    @pl.when(s < n // rows_per)
    def _():
        pltpu.sync_copy(u_acc.at[pl.ds(r0, rows_per)],
                        u2.at[c, pl.ds(r0, rows_per)])


@functools.partial(jax.jit, static_argnums=(0,))
def _sc_edge(heads, zelr, era, c2, src, dst, z136):
    n = zelr.shape[0]
    e = src.shape[0]
    mesh = plsc.VectorSubcoreMesh(core_axis_name="c", subcore_axis_name="s",
                                  num_cores=2, num_subcores=16)
    body = functools.partial(_sc_edge_body, heads, n, e)
    f = pl.kernel(
        body,
        out_type=[
            jax.ShapeDtypeStruct((2, n, 136), jnp.float32),
        ],
        mesh=mesh,
        compiler_params=pltpu.CompilerParams(use_tc_tiling_on_sc=False, needs_layout_passes=False),
        scratch_types=[
            pltpu.VMEM_SHARED((n, 136), jnp.float32),
            pltpu.VMEM((_B,), jnp.int32),
            pltpu.VMEM((_B,), jnp.int32),
            pltpu.VMEM((_B,), jnp.int32),
            pltpu.VMEM((_B,), jnp.int32),
            pltpu.VMEM((_B, 136), jnp.float32),
            pltpu.VMEM((_B, 136), jnp.float32),
            pltpu.VMEM((_B, 8), jnp.float32),
            pltpu.VMEM((_B, 8), jnp.float32),
            pltpu.VMEM((_B, 136), jnp.float32),
            pltpu.VMEM((_B, 136), jnp.float32),
            pltpu.VMEM((_B,), jnp.int32),
            pltpu.VMEM((_B,), jnp.int32),
            pltpu.VMEM((16,), jnp.float32),
            pltpu.SemaphoreType.DMA,
            pltpu.SemaphoreType.DMA,
            pltpu.SemaphoreType.DMA,
            pltpu.SemaphoreType.DMA,
            pltpu.SemaphoreType.DMA,
            pltpu.SemaphoreType.DMA,
        ],
    )
    return f(zelr, era, c2, src, dst, z136)


def _expand_att(a):
    heads, outd = a.shape
    k = jnp.arange(128)
    m = (k[:, None] // outd == jnp.arange(8)[None, :]).astype(jnp.float32)
    return m * a.reshape(-1)[:, None]


def _expand_p(outd):
    return (jnp.arange(8)[:, None] == (jnp.arange(128)[None, :] // outd)
            ).astype(jnp.float32)


def kernel(feature, edge_index, W_emb, b_emb, W1, al1, ar1, g1, bt1,
           W2, al2, ar2, g2, bt2, W3, al3, ar3, g3, bt3,
           W4, al4, ar4, g4, bt4):
    n = feature.shape[0]
    src = edge_index[0]
    dst = edge_index[1]
    z136 = jnp.zeros((1000, 136), jnp.float32)
    bn_scale = 1.0 / jnp.sqrt(1.0 + BN_EPS)

    layers = [(W1, al1, ar1, g1, bt1, 8), (W2, al2, ar2, g2, bt2, 8),
              (W3, al3, ar3, g3, bt3, 8), (W4, al4, ar4, g4, bt4, 1)]
    zelr, era, cm, h = _tc_first(feature, W_emb, b_emb.reshape(1, 128),
                                 W1, _expand_att(al1), _expand_att(ar1))
    for i, (w, al, ar, g, bt, heads) in enumerate(layers):
        outd = 128 // heads
        cc = cm[0] + cm[1]
        shift = jnp.maximum(cc, 0.2 * cc)
        c2 = jnp.tile(shift, 2)
        (u2,) = _sc_edge(heads, zelr, era, c2, src, dst, z136)
        gs = (g * bn_scale).reshape(1, 128)
        btr = bt.reshape(1, 128)
        if i < 3:
            nw, nal, nar, _, _, _ = layers[i + 1]
            zelr, era, cm, h = _tc_mid(u2, _expand_p(outd), gs, btr, h,
                                       nw, _expand_att(nal), _expand_att(nar))
        else:
            h = _tc_last(u2, _expand_p(outd), gs, btr, h)
    return h
